# Initial kernel scaffold; baseline (speedup 1.0000x reference)
import jax
import jax.numpy as jnp
from jax.experimental import pallas as pl


def kernel(x, codebook):
    raise NotImplementedError("write your pallas kernel here")


# speedup vs baseline: 1.9826x; 1.9826x over previous
import jax
import jax.numpy as jnp
from jax.experimental import pallas as pl

_N_COMPONENTS = 1024
_EMBEDDING_DIM = 64
_BETA = 0.25
_BLK = 512


def _vq_block(x_ref, cb_ref, soft_ref, q_ref, loss_ref):
    x = x_ref[...]
    cb = cb_ref[...]
    sim = jnp.dot(x, cb, preferred_element_type=jnp.float32)
    x2 = jnp.sum(x * x, axis=1, keepdims=True)
    c2 = jnp.sum(cb * cb, axis=0, keepdims=True)
    dist = x2 + c2 - 2.0 * sim
    s = (1.0 / dist) ** 2
    soft_ref[...] = s / jnp.sum(s, axis=1, keepdims=True)
    idx = jnp.argmin(dist, axis=1)
    enc = (jax.lax.broadcasted_iota(jnp.int32, (_BLK, _N_COMPONENTS), 1)
           == idx[:, None]).astype(jnp.float32)
    q = jax.lax.dot_general(enc, cb,
                            dimension_numbers=(((1,), (1,)), ((), ())),
                            preferred_element_type=jnp.float32)
    q_ref[...] = q
    diff = q - x
    sse = jnp.sum(diff * diff).reshape(1, 1)

    @pl.when(pl.program_id(0) == 0)
    def _init():
        loss_ref[...] = jnp.zeros_like(loss_ref)

    loss_ref[...] += sse


def kernel(x, codebook):
    input_shape = x.shape
    flat = x.reshape(-1, _EMBEDDING_DIM)
    rows = flat.shape[0]
    grid = rows // _BLK

    soft, q, loss = pl.pallas_call(
        _vq_block,
        grid=(grid,),
        in_specs=[
            pl.BlockSpec((_BLK, _EMBEDDING_DIM), lambda i: (i, 0)),
            pl.BlockSpec((_EMBEDDING_DIM, _N_COMPONENTS), lambda i: (0, 0)),
        ],
        out_specs=[
            pl.BlockSpec((_BLK, _N_COMPONENTS), lambda i: (i, 0)),
            pl.BlockSpec((_BLK, _EMBEDDING_DIM), lambda i: (i, 0)),
            pl.BlockSpec((1, 1), lambda i: (0, 0)),
        ],
        out_shape=[
            jax.ShapeDtypeStruct((rows, _N_COMPONENTS), jnp.float32),
            jax.ShapeDtypeStruct((rows, _EMBEDDING_DIM), jnp.float32),
            jax.ShapeDtypeStruct((1, 1), jnp.float32),
        ],
    )(flat, codebook)

    quantized = q.reshape(input_shape)
    vq_loss = (1.0 + _BETA) * loss[0, 0] / flat.size
    return quantized, soft, vq_loss


# speedup vs baseline: 2.0721x; 1.0452x over previous
import jax
import jax.numpy as jnp
from jax.experimental import pallas as pl

_N_COMPONENTS = 1024
_EMBEDDING_DIM = 64
_BETA = 0.25
_BLK = 1024


def _vq_block(x_ref, cb_ref, soft_ref, q_ref, loss_ref):
    x = x_ref[...]
    cb = cb_ref[...]
    sim = jnp.dot(x, cb, preferred_element_type=jnp.float32)
    x2 = jnp.sum(x * x, axis=1, keepdims=True)
    c2 = jnp.sum(cb * cb, axis=0, keepdims=True)
    dist = x2 + c2 - 2.0 * sim
    s = (1.0 / dist) ** 2
    soft_ref[...] = s / jnp.sum(s, axis=1, keepdims=True)
    idx = jnp.argmin(dist, axis=1)
    enc = (jax.lax.broadcasted_iota(jnp.int32, (_BLK, _N_COMPONENTS), 1)
           == idx[:, None]).astype(jnp.float32)
    q = jax.lax.dot_general(enc, cb,
                            dimension_numbers=(((1,), (1,)), ((), ())),
                            preferred_element_type=jnp.float32)
    q_ref[...] = q
    diff = q - x
    sse = jnp.sum(diff * diff).reshape(1, 1)

    @pl.when(pl.program_id(0) == 0)
    def _init():
        loss_ref[...] = jnp.zeros_like(loss_ref)

    loss_ref[...] += sse


def kernel(x, codebook):
    input_shape = x.shape
    flat = x.reshape(-1, _EMBEDDING_DIM)
    rows = flat.shape[0]
    grid = rows // _BLK

    soft, q, loss = pl.pallas_call(
        _vq_block,
        grid=(grid,),
        in_specs=[
            pl.BlockSpec((_BLK, _EMBEDDING_DIM), lambda i: (i, 0)),
            pl.BlockSpec((_EMBEDDING_DIM, _N_COMPONENTS), lambda i: (0, 0)),
        ],
        out_specs=[
            pl.BlockSpec((_BLK, _N_COMPONENTS), lambda i: (i, 0)),
            pl.BlockSpec((_BLK, _EMBEDDING_DIM), lambda i: (i, 0)),
            pl.BlockSpec((1, 1), lambda i: (0, 0)),
        ],
        out_shape=[
            jax.ShapeDtypeStruct((rows, _N_COMPONENTS), jnp.float32),
            jax.ShapeDtypeStruct((rows, _EMBEDDING_DIM), jnp.float32),
            jax.ShapeDtypeStruct((1, 1), jnp.float32),
        ],
    )(flat, codebook)

    quantized = q.reshape(input_shape)
    vq_loss = (1.0 + _BETA) * loss[0, 0] / flat.size
    return quantized, soft, vq_loss


# speedup vs baseline: 2.1282x; 1.0271x over previous
import jax
import jax.numpy as jnp
from jax.experimental import pallas as pl

_N_COMPONENTS = 1024
_EMBEDDING_DIM = 64
_BETA = 0.25
_BLK = 2304


def _vq_block(x_ref, cb_ref, soft_ref, q_ref, loss_ref):
    x = x_ref[...]
    cb = cb_ref[...]
    sim = jnp.dot(x, cb, preferred_element_type=jnp.float32)
    x2 = jnp.sum(x * x, axis=1, keepdims=True)
    c2 = jnp.sum(cb * cb, axis=0, keepdims=True)
    dist = x2 + c2 - 2.0 * sim
    s = (1.0 / dist) ** 2
    soft_ref[...] = s / jnp.sum(s, axis=1, keepdims=True)
    idx = jnp.argmin(dist, axis=1)
    enc = (jax.lax.broadcasted_iota(jnp.int32, (_BLK, _N_COMPONENTS), 1)
           == idx[:, None]).astype(jnp.float32)
    q = jax.lax.dot_general(enc, cb,
                            dimension_numbers=(((1,), (1,)), ((), ())),
                            preferred_element_type=jnp.float32)
    q_ref[...] = q
    diff = q - x
    sse = jnp.sum(diff * diff).reshape(1, 1)

    @pl.when(pl.program_id(0) == 0)
    def _init():
        loss_ref[...] = jnp.zeros_like(loss_ref)

    loss_ref[...] += sse


def kernel(x, codebook):
    input_shape = x.shape
    flat = x.reshape(-1, _EMBEDDING_DIM)
    rows = flat.shape[0]
    grid = rows // _BLK

    soft, q, loss = pl.pallas_call(
        _vq_block,
        grid=(grid,),
        in_specs=[
            pl.BlockSpec((_BLK, _EMBEDDING_DIM), lambda i: (i, 0)),
            pl.BlockSpec((_EMBEDDING_DIM, _N_COMPONENTS), lambda i: (0, 0)),
        ],
        out_specs=[
            pl.BlockSpec((_BLK, _N_COMPONENTS), lambda i: (i, 0)),
            pl.BlockSpec((_BLK, _EMBEDDING_DIM), lambda i: (i, 0)),
            pl.BlockSpec((1, 1), lambda i: (0, 0)),
        ],
        out_shape=[
            jax.ShapeDtypeStruct((rows, _N_COMPONENTS), jnp.float32),
            jax.ShapeDtypeStruct((rows, _EMBEDDING_DIM), jnp.float32),
            jax.ShapeDtypeStruct((1, 1), jnp.float32),
        ],
    )(flat, codebook)

    quantized = q.reshape(input_shape)
    vq_loss = (1.0 + _BETA) * loss[0, 0] / flat.size
    return quantized, soft, vq_loss


# speedup vs baseline: 2.1735x; 1.0213x over previous
import jax
import jax.numpy as jnp
from jax.experimental import pallas as pl
from jax.experimental.pallas import tpu as pltpu

_N_COMPONENTS = 1024
_EMBEDDING_DIM = 64
_BETA = 0.25
_BLK = 2304


def _vq_block(x_ref, cb_ref, soft_ref, q_ref, loss_ref, c2_ref):
    @pl.when(pl.program_id(0) == 0)
    def _prologue():
        cb0 = cb_ref[...]
        c2_ref[...] = jnp.sum(cb0 * cb0, axis=0, keepdims=True)
        loss_ref[...] = jnp.zeros_like(loss_ref)

    x = x_ref[...]
    cb = cb_ref[...]
    sim = jnp.dot(x, cb, preferred_element_type=jnp.float32)
    x2 = jnp.sum(x * x, axis=1, keepdims=True)
    dist = x2 + c2_ref[...] - 2.0 * sim
    s = (1.0 / dist) ** 2
    soft_ref[...] = s / jnp.sum(s, axis=1, keepdims=True)
    idx = jnp.argmin(dist, axis=1)
    enc = (jax.lax.broadcasted_iota(jnp.int32, (_BLK, _N_COMPONENTS), 1)
           == idx[:, None]).astype(jnp.float32)
    q = jax.lax.dot_general(enc, cb,
                            dimension_numbers=(((1,), (1,)), ((), ())),
                            preferred_element_type=jnp.float32)
    q_ref[...] = q
    mind = jnp.min(dist, axis=1)
    loss_ref[...] += jnp.sum(mind).reshape(1, 1)


def kernel(x, codebook):
    input_shape = x.shape
    flat = x.reshape(-1, _EMBEDDING_DIM)
    rows = flat.shape[0]
    grid = rows // _BLK

    soft, q, loss = pl.pallas_call(
        _vq_block,
        grid=(grid,),
        in_specs=[
            pl.BlockSpec((_BLK, _EMBEDDING_DIM), lambda i: (i, 0)),
            pl.BlockSpec((_EMBEDDING_DIM, _N_COMPONENTS), lambda i: (0, 0)),
        ],
        out_specs=[
            pl.BlockSpec((_BLK, _N_COMPONENTS), lambda i: (i, 0)),
            pl.BlockSpec((_BLK, _EMBEDDING_DIM), lambda i: (i, 0)),
            pl.BlockSpec((1, 1), lambda i: (0, 0)),
        ],
        out_shape=[
            jax.ShapeDtypeStruct((rows, _N_COMPONENTS), jnp.float32),
            jax.ShapeDtypeStruct((rows, _EMBEDDING_DIM), jnp.float32),
            jax.ShapeDtypeStruct((1, 1), jnp.float32),
        ],
        scratch_shapes=[pltpu.VMEM((1, _N_COMPONENTS), jnp.float32)],
    )(flat, codebook)

    quantized = q.reshape(input_shape)
    vq_loss = (1.0 + _BETA) * loss[0, 0] / flat.size
    return quantized, soft, vq_loss
